# Initial kernel scaffold; baseline (speedup 1.0000x reference)
import jax
import jax.numpy as jnp
from jax.experimental import pallas as pl


def kernel(x, w1, b1, w2, b2):
    raise NotImplementedError("write your pallas kernel here")


# speedup vs baseline: 1.1948x; 1.1948x over previous
import functools

import jax
import jax.numpy as jnp
from jax.experimental import pallas as pl
from jax.experimental.pallas import tpu as pltpu


def _body(x_ref, w1_ref, b1_ref, w2_ref, b2_ref, o_ref, acc_ref, *, inv_s, n_seq):
    k = pl.program_id(1)

    part = jnp.sum(x_ref[...].astype(jnp.float32), axis=1)

    @pl.when(k == 0)
    def _init():
        acc_ref[...] = part

    @pl.when(k > 0)
    def _accum():
        acc_ref[...] += part

    @pl.when(k == n_seq - 1)
    def _epilogue():
        mean = acc_ref[...] * inv_s
        h = jnp.dot(mean, w1_ref[...], preferred_element_type=jnp.float32)
        h = jnp.maximum(h + b1_ref[...], 0.0)
        logits = jnp.dot(h, w2_ref[...], preferred_element_type=jnp.float32)
        logits = logits + b2_ref[...]
        m = jnp.max(logits, axis=-1, keepdims=True)
        z = logits - m
        o_ref[...] = z - jnp.log(jnp.sum(jnp.exp(z), axis=-1, keepdims=True))


def _largest_divisor_leq(n, cap, step=8):
    best = None
    for d in range(step, min(n, cap) + 1, step):
        if n % d == 0:
            best = d
    return best


def kernel(x, w1, b1, w2, b2):
    B, S, H = x.shape
    L = w2.shape[1]
    b1 = jnp.asarray(b1, jnp.float32).reshape(1, H)
    b2 = jnp.asarray(b2, jnp.float32).reshape(1, L)

    L_pad = -(-L // 128) * 128
    if L_pad != L:
        w2 = jnp.pad(w2, ((0, 0), (0, L_pad - L)))
        b2 = jnp.pad(b2, ((0, 0), (0, L_pad - L)), constant_values=-1e30)

    block_b = _largest_divisor_leq(B, -(-B // 2)) or B
    grid_b = B // block_b

    block_s = _largest_divisor_leq(S, 32) or S
    grid_k = S // block_s

    body = functools.partial(_body, inv_s=1.0 / S, n_seq=grid_k)

    out = pl.pallas_call(
        body,
        out_shape=jax.ShapeDtypeStruct((B, L_pad), jnp.float32),
        grid=(grid_b, grid_k),
        in_specs=[
            pl.BlockSpec((block_b, block_s, H), lambda b, k: (b, k, 0)),
            pl.BlockSpec((H, H), lambda b, k: (0, 0)),
            pl.BlockSpec((1, H), lambda b, k: (0, 0)),
            pl.BlockSpec((H, L_pad), lambda b, k: (0, 0)),
            pl.BlockSpec((1, L_pad), lambda b, k: (0, 0)),
        ],
        out_specs=pl.BlockSpec((block_b, L_pad), lambda b, k: (b, 0)),
        scratch_shapes=[pltpu.VMEM((block_b, H), jnp.float32)],
        compiler_params=pltpu.CompilerParams(
            dimension_semantics=("parallel", "arbitrary"),
        ),
    )(x, w1, b1, w2, b2)

    return {"preds_length": out[:, :L]}
